# baseline (speedup 1.0000x reference)
import functools

import jax
import jax.numpy as jnp
from jax import lax
from jax.experimental import pallas as pl
from jax.experimental.pallas import tpu as pltpu

TOPK_SMALL = 1
TOPK_LARGE = 5

BAT_BLK = 128
VOC_BLK = 50000


def _body(n_bat, n_voc, cb_last, tref, *refs):
    wins = refs[:BAT_BLK]
    tgt_ref = refs[BAT_BLK]
    x_ref = refs[BAT_BLK + 1]
    out_ref = refs[BAT_BLK + 2]
    acc_ref = refs[BAT_BLK + 3]
    tvs_ref = refs[BAT_BLK + 4]

    rb = pl.program_id(0)
    cb = pl.program_id(1)
    tg = tgt_ref[...]

    @pl.when(cb == 0)
    def _():
        acc_ref[...] = jnp.zeros_like(acc_ref)
        lane = lax.broadcasted_iota(jnp.int32, (8, BAT_BLK), 1)
        srow = lax.broadcasted_iota(jnp.int32, (8, BAT_BLK), 0)
        w = jnp.zeros((8, BAT_BLK), jnp.float32)
        for r in range(BAT_BLK):
            w = w + jnp.where(lane == r, wins[r][...], 0.0)
        sel = srow == jnp.bitwise_and(tg, 7)
        tvs_ref[...] = jnp.sum(jnp.where(sel, w, 0.0), axis=0, keepdims=True)

    x = x_ref[...]
    tv = tvs_ref[...]
    vidx = cb * VOC_BLK + lax.broadcasted_iota(
        jnp.int32, (VOC_BLK, BAT_BLK), 0)
    valid = vidx < n_voc
    better = ((x > tv) | ((x == tv) & (vidx < tg))) & valid
    acc_ref[...] += jnp.sum(
        better.astype(jnp.float32), axis=0, keepdims=True)

    @pl.when(cb == cb_last)
    def _():
        a = acc_ref[...]
        inv_b = 1.0 / n_bat
        s1 = jnp.sum((a < TOPK_SMALL).astype(jnp.float32)) * inv_b
        s5 = jnp.sum((a < TOPK_LARGE).astype(jnp.float32)) * inv_b
        lane = lax.broadcasted_iota(jnp.int32, (1, 128), 1)
        contrib = (jnp.where(lane == 0, s1, 0.0)
                   + jnp.where(lane == 1, s5, 0.0))

        @pl.when(rb == 0)
        def _():
            out_ref[...] = contrib

        @pl.when(rb > 0)
        def _():
            out_ref[...] += contrib


def _win_spec(r):
    return pl.BlockSpec(
        (8, BAT_BLK),
        lambda rb, cb, tref: (tref[BAT_BLK * rb + r] // 8, rb))


def kernel(output, target):
    n_bat, n_voc = output.shape
    tgt = target.astype(jnp.int32)
    xt = output.T

    rb_n = n_bat // BAT_BLK
    cb_n = (n_voc + VOC_BLK - 1) // VOC_BLK
    out = pl.pallas_call(
        functools.partial(_body, n_bat, n_voc, cb_n - 1),
        grid_spec=pltpu.PrefetchScalarGridSpec(
            num_scalar_prefetch=1,
            grid=(rb_n, cb_n),
            in_specs=(
                [_win_spec(r) for r in range(BAT_BLK)]
                + [
                    pl.BlockSpec((1, BAT_BLK), lambda rb, cb, tref: (0, rb)),
                    pl.BlockSpec((VOC_BLK, BAT_BLK),
                                 lambda rb, cb, tref: (cb, rb)),
                ]
            ),
            out_specs=pl.BlockSpec((1, 128), lambda rb, cb, tref: (0, 0)),
            scratch_shapes=[
                pltpu.VMEM((1, BAT_BLK), jnp.float32),
                pltpu.VMEM((1, BAT_BLK), jnp.float32),
            ],
        ),
        out_shape=jax.ShapeDtypeStruct((1, 128), jnp.float32),
    )(tgt, *([xt] * BAT_BLK), tgt.reshape(1, n_bat), xt)
    return out[0, :2]


# speedup vs baseline: 1.4477x; 1.4477x over previous
import functools

import jax
import jax.numpy as jnp
from jax import lax
from jax.experimental import pallas as pl
from jax.experimental.pallas import tpu as pltpu

TOPK_SMALL = 1
TOPK_LARGE = 5

NWIN = 128
BAT_BLK = 128
VOC_BLK = 50000


def _gather_body(tref, *refs):
    xs = refs[:NWIN]
    win_ref = refs[NWIN]
    lane = lax.broadcasted_iota(jnp.int32, (8, NWIN), 1)
    acc = jnp.zeros((8, NWIN), jnp.float32)
    for r in range(NWIN):
        acc = acc + jnp.where(lane == r, xs[r][...], 0.0)
    win_ref[...] = acc


def _win_spec(r):
    return pl.BlockSpec(
        (8, NWIN), lambda i, tref: (tref[NWIN * i + r] // 8, i))


def _count_body(n_bat, n_voc, cb_last, win_ref, tgt_ref, x_ref,
                out_ref, acc_ref, tvs_ref):
    rb = pl.program_id(0)
    cb = pl.program_id(1)
    tg = tgt_ref[...]

    @pl.when(cb == 0)
    def _():
        acc_ref[...] = jnp.zeros_like(acc_ref)
        srow = lax.broadcasted_iota(jnp.int32, (8, BAT_BLK), 0)
        sel = srow == jnp.bitwise_and(tg, 7)
        tvs_ref[...] = jnp.sum(
            jnp.where(sel, win_ref[...], 0.0), axis=0, keepdims=True)

    x = x_ref[...]
    tv = tvs_ref[...]
    vidx = cb * VOC_BLK + lax.broadcasted_iota(
        jnp.int32, (VOC_BLK, BAT_BLK), 0)
    better = (x > tv) | ((x == tv) & (vidx < tg))
    if n_voc % VOC_BLK != 0:
        better &= vidx < n_voc
    acc_ref[...] += jnp.sum(
        better.astype(jnp.float32), axis=0, keepdims=True)

    @pl.when(cb == cb_last)
    def _():
        a = acc_ref[...]
        inv_b = 1.0 / n_bat
        s1 = jnp.sum((a < TOPK_SMALL).astype(jnp.float32)) * inv_b
        s5 = jnp.sum((a < TOPK_LARGE).astype(jnp.float32)) * inv_b
        lane = lax.broadcasted_iota(jnp.int32, (1, 128), 1)
        contrib = (jnp.where(lane == 0, s1, 0.0)
                   + jnp.where(lane == 1, s5, 0.0))

        @pl.when(rb == 0)
        def _():
            out_ref[...] = contrib

        @pl.when(rb > 0)
        def _():
            out_ref[...] += contrib


def kernel(output, target):
    n_bat, n_voc = output.shape
    tgt = target.astype(jnp.int32)
    xt = output.T

    win = pl.pallas_call(
        _gather_body,
        grid_spec=pltpu.PrefetchScalarGridSpec(
            num_scalar_prefetch=1,
            grid=(n_bat // NWIN,),
            in_specs=[_win_spec(r) for r in range(NWIN)],
            out_specs=pl.BlockSpec((8, NWIN), lambda i, tref: (0, i)),
        ),
        out_shape=jax.ShapeDtypeStruct((8, n_bat), jnp.float32),
    )(tgt, *([xt] * NWIN))

    rb_n = n_bat // BAT_BLK
    cb_n = (n_voc + VOC_BLK - 1) // VOC_BLK
    out = pl.pallas_call(
        functools.partial(_count_body, n_bat, n_voc, cb_n - 1),
        grid=(rb_n, cb_n),
        in_specs=[
            pl.BlockSpec((8, BAT_BLK), lambda rb, cb: (0, rb)),
            pl.BlockSpec((1, BAT_BLK), lambda rb, cb: (0, rb)),
            pl.BlockSpec((VOC_BLK, BAT_BLK), lambda rb, cb: (cb, rb)),
        ],
        out_specs=pl.BlockSpec((1, 128), lambda rb, cb: (0, 0)),
        out_shape=jax.ShapeDtypeStruct((1, 128), jnp.float32),
        scratch_shapes=[
            pltpu.VMEM((1, BAT_BLK), jnp.float32),
            pltpu.VMEM((1, BAT_BLK), jnp.float32),
        ],
    )(win, tgt.reshape(1, n_bat), xt)
    return out[0, :2]
